# baseline (speedup 1.0000x reference)
import functools

import jax
import jax.numpy as jnp
from jax.experimental import pallas as pl
from jax.experimental.pallas import tpu as pltpu

_NUM_SELECT = 300
_NUM_CLASSES = 91
_SEG_ROWS = 10
_NUM_SEGS = 64
_BIG = 1 << 30


def _body(logits_ref, boxes_ref, sizes_ref, boxes_out_ref, scores_ref,
          labels_ref, scratch_ref):
    num_queries = boxes_ref.shape[1]

    prob = jax.nn.sigmoid(logits_ref[0])
    scratch_ref[...] = prob

    seg_max = jnp.max(
        jnp.max(prob.reshape(_NUM_SEGS, _SEG_ROWS, 128), axis=1),
        axis=1, keepdims=True)
    seg_iota = jax.lax.broadcasted_iota(jnp.int32, (_NUM_SEGS, 1), 0)
    out_iota = jax.lax.broadcasted_iota(jnp.int32, (1, _NUM_SELECT), 1)
    local_iota = (
        jax.lax.broadcasted_iota(jnp.int32, (_SEG_ROWS, 128), 0) * 128
        + jax.lax.broadcasted_iota(jnp.int32, (_SEG_ROWS, 128), 1))

    def step(i, carry):
        scores, idx, seg_max = carry
        m = jnp.max(seg_max)
        s = jnp.min(jnp.where(seg_max == m, seg_iota, _NUM_SEGS))
        blk = scratch_ref[pl.ds(s * _SEG_ROWS, _SEG_ROWS), :]
        flat = local_iota + s * (_SEG_ROWS * 128)
        j = jnp.min(jnp.where(blk == m, flat, _BIG))
        blk = jnp.where(flat == j, -1.0, blk)
        scratch_ref[pl.ds(s * _SEG_ROWS, _SEG_ROWS), :] = blk
        seg_max = jnp.where(seg_iota == s, jnp.max(blk), seg_max)
        scores = jnp.where(out_iota == i, m, scores)
        idx = jnp.where(out_iota == i, j, idx)
        return scores, idx, seg_max

    scores = jnp.zeros((1, _NUM_SELECT), jnp.float32)
    idx = jnp.zeros((1, _NUM_SELECT), jnp.int32)
    scores, idx, _ = jax.lax.fori_loop(
        0, _NUM_SELECT, step, (scores, idx, seg_max))

    scores_ref[...] = scores[None]
    labels_ref[...] = (idx % _NUM_CLASSES)[None]
    qidx = idx // _NUM_CLASSES

    onehot = (
        qidx[0][:, None]
        == jax.lax.broadcasted_iota(jnp.int32, (_NUM_SELECT, num_queries), 1)
    ).astype(jnp.float32)
    cxcywh = jnp.dot(onehot, boxes_ref[0],
                     preferred_element_type=jnp.float32)
    cx = cxcywh[:, 0:1]
    cy = cxcywh[:, 1:2]
    w = cxcywh[:, 2:3]
    h = cxcywh[:, 3:4]
    xyxy = jnp.concatenate(
        [cx - 0.5 * w, cy - 0.5 * h, cx + 0.5 * w, cy + 0.5 * h], axis=1)

    ts = sizes_ref[0]
    scale = jnp.concatenate(
        [ts[:, 1:2], ts[:, 0:1], ts[:, 1:2], ts[:, 0:1]], axis=1
    ).astype(jnp.float32)
    boxes_out_ref[...] = (xyxy * scale)[None]


def kernel(pred_logits, pred_boxes, target_sizes, img_metas):
    del img_metas
    b, q, c = pred_logits.shape
    rows = _NUM_SEGS * _SEG_ROWS
    flat = pred_logits.reshape(b, q * c)
    pad = rows * 128 - q * c
    flat = jnp.pad(flat, ((0, 0), (0, pad)), constant_values=-1e9)
    flat = flat.reshape(b, rows, 128)
    sizes = target_sizes.reshape(b, 1, 2)
    boxes, scores, labels = pl.pallas_call(
        _body,
        grid=(b,),
        in_specs=[
            pl.BlockSpec((1, rows, 128), lambda i: (i, 0, 0)),
            pl.BlockSpec((1, q, 4), lambda i: (i, 0, 0)),
            pl.BlockSpec((1, 1, 2), lambda i: (i, 0, 0)),
        ],
        out_specs=[
            pl.BlockSpec((1, _NUM_SELECT, 4), lambda i: (i, 0, 0)),
            pl.BlockSpec((1, 1, _NUM_SELECT), lambda i: (i, 0, 0)),
            pl.BlockSpec((1, 1, _NUM_SELECT), lambda i: (i, 0, 0)),
        ],
        out_shape=[
            jax.ShapeDtypeStruct((b, _NUM_SELECT, 4), jnp.float32),
            jax.ShapeDtypeStruct((b, 1, _NUM_SELECT), jnp.float32),
            jax.ShapeDtypeStruct((b, 1, _NUM_SELECT), jnp.int32),
        ],
        scratch_shapes=[pltpu.VMEM((rows, 128), jnp.float32)],
        compiler_params=pltpu.CompilerParams(
            dimension_semantics=("arbitrary",)),
    )(flat, pred_boxes, sizes)
    return boxes, scores.reshape(b, _NUM_SELECT), labels.reshape(b, _NUM_SELECT)
